# baseline (speedup 1.0000x reference)
import functools

import jax
import jax.numpy as jnp
from jax import lax
from jax.experimental import pallas as pl
from jax.experimental.pallas import tpu as pltpu
from jax.experimental.pallas import tpu_sc as plsc

B = 16384
D = 64
NUM_CORES = 2
NUM_SUBCORES = 16
NW = NUM_CORES * NUM_SUBCORES
BPW = B // NW
L = 16
GROUPS = BPW // L

_mesh = plsc.VectorSubcoreMesh(core_axis_name="c", subcore_axis_name="s")


@functools.partial(
    pl.kernel,
    out_type=(
        jax.ShapeDtypeStruct((B,), jnp.float32),
        jax.ShapeDtypeStruct((B,), jnp.float32),
    ),
    mesh=_mesh,
    scratch_types=[
        pltpu.VMEM((BPW,), jnp.int32),
        pltpu.VMEM((BPW,), jnp.int32),
        pltpu.VMEM((BPW,), jnp.int32),
        pltpu.VMEM((BPW, D), jnp.float32),
        pltpu.VMEM((BPW, D), jnp.float32),
        pltpu.VMEM((BPW, D), jnp.float32),
        pltpu.VMEM((BPW,), jnp.float32),
        pltpu.VMEM((BPW,), jnp.float32),
        pltpu.SemaphoreType.DMA,
    ],
    compiler_params=pltpu.CompilerParams(
        needs_layout_passes=False, use_tc_tiling_on_sc=False),
)
def _mfbpr(user_h, item_i_h, item_j_h, eu_h, ei_h, oi_h, oj_h,
           idx_u, idx_i, idx_j, u_v, vi_v, vj_v, oi_v, oj_v, sem):
    wid = lax.axis_index("s") * NUM_CORES + lax.axis_index("c")
    base = wid * BPW
    pltpu.sync_copy(user_h.at[pl.ds(base, BPW)], idx_u)
    pltpu.sync_copy(item_i_h.at[pl.ds(base, BPW)], idx_i)
    pltpu.sync_copy(item_j_h.at[pl.ds(base, BPW)], idx_j)
    cu = pltpu.async_copy(eu_h.at[idx_u], u_v, sem)
    ci = pltpu.async_copy(ei_h.at[idx_i], vi_v, sem)
    cj = pltpu.async_copy(ei_h.at[idx_j], vj_v, sem)
    cu.wait()
    ci.wait()
    cj.wait()

    lanes = lax.iota(jnp.int32, L)

    def group_body(g, carry):
        rows = g * L + lanes
        acc_i = jnp.zeros((L,), jnp.float32)
        acc_j = jnp.zeros((L,), jnp.float32)
        for k in range(D):
            col = jnp.full((L,), k, dtype=jnp.int32)
            u = plsc.load_gather(u_v, [rows, col])
            vi = plsc.load_gather(vi_v, [rows, col])
            vj = plsc.load_gather(vj_v, [rows, col])
            acc_i = acc_i + u * vi
            acc_j = acc_j + u * vj
        oi_v[pl.ds(g * L, L)] = acc_i
        oj_v[pl.ds(g * L, L)] = acc_j
        return carry

    lax.fori_loop(0, GROUPS, group_body, 0)

    pltpu.sync_copy(oi_v, oi_h.at[pl.ds(base, BPW)])
    pltpu.sync_copy(oj_v, oj_h.at[pl.ds(base, BPW)])


def kernel(user, item_i, item_j, embed_user, embed_item):
    return _mfbpr(user.astype(jnp.int32), item_i.astype(jnp.int32),
                  item_j.astype(jnp.int32), embed_user, embed_item)


# speedup vs baseline: 1.4085x; 1.4085x over previous
import functools

import jax
import jax.numpy as jnp
from jax import lax
from jax.experimental import pallas as pl
from jax.experimental.pallas import tpu as pltpu
from jax.experimental.pallas import tpu_sc as plsc

B = 16384
D = 64
NUM_CORES = 2
NUM_SUBCORES = 16
NW = NUM_CORES * NUM_SUBCORES
BPW = B // NW
L = 16
C = 64
K = 4


@functools.partial(
    pl.kernel,
    out_type=(
        jax.ShapeDtypeStruct((B,), jnp.float32),
        jax.ShapeDtypeStruct((B,), jnp.float32),
    ),
    mesh=plsc.VectorSubcoreMesh(core_axis_name="c", subcore_axis_name="s"),
    scratch_types=[
        pltpu.VMEM((BPW,), jnp.int32),
        pltpu.VMEM((BPW,), jnp.int32),
        pltpu.VMEM((BPW,), jnp.int32),
        pltpu.VMEM((C, D), jnp.float32),
        pltpu.VMEM((C, D), jnp.float32),
        pltpu.VMEM((C, D), jnp.float32),
        pltpu.VMEM((BPW,), jnp.float32),
        pltpu.VMEM((BPW,), jnp.float32),
        pltpu.SemaphoreType.DMA,
    ],
    compiler_params=pltpu.CompilerParams(needs_layout_passes=False),
)
def _mfbpr(user_h, item_i_h, item_j_h, eu_h, ei_h, oi_h, oj_h,
           idx_u, idx_i, idx_j, u_v, vi_v, vj_v, oi_v, oj_v, sem):
    wid = lax.axis_index("s") * NUM_CORES + lax.axis_index("c")
    base = wid * BPW
    pltpu.sync_copy(user_h.at[pl.ds(base, BPW)], idx_u)
    pltpu.sync_copy(item_i_h.at[pl.ds(base, BPW)], idx_i)
    pltpu.sync_copy(item_j_h.at[pl.ds(base, BPW)], idx_j)

    lanes = lax.iota(jnp.int32, L)

    def chunk_body(c, carry):
        cbase = c * C
        for g in range(C // L):
            rbase = g * L
            iu = idx_u[pl.ds(cbase + rbase, L)]
            ii = idx_i[pl.ds(cbase + rbase, L)]
            ij = idx_j[pl.ds(cbase + rbase, L)]
            for s in range(L // K):
                copies = []
                for t in range(K):
                    lane = s * K + t
                    r = rbase + lane
                    copies.append(
                        pltpu.async_copy(eu_h.at[iu[lane]], u_v.at[r], sem))
                    copies.append(
                        pltpu.async_copy(ei_h.at[ii[lane]], vi_v.at[r], sem))
                    copies.append(
                        pltpu.async_copy(ei_h.at[ij[lane]], vj_v.at[r], sem))
                for cp in copies:
                    cp.wait()
        for g in range(C // L):
            rows = g * L + lanes
            acc_i = jnp.zeros((L,), jnp.float32)
            acc_j = jnp.zeros((L,), jnp.float32)
            for k in range(D):
                col = jnp.full((L,), k, dtype=jnp.int32)
                u = plsc.load_gather(u_v, [rows, col])
                vi = plsc.load_gather(vi_v, [rows, col])
                vj = plsc.load_gather(vj_v, [rows, col])
                acc_i = acc_i + u * vi
                acc_j = acc_j + u * vj
            oi_v[pl.ds(cbase + g * L, L)] = acc_i
            oj_v[pl.ds(cbase + g * L, L)] = acc_j
        return carry

    lax.fori_loop(0, BPW // C, chunk_body, 0)

    pltpu.sync_copy(oi_v, oi_h.at[pl.ds(base, BPW)])
    pltpu.sync_copy(oj_v, oj_h.at[pl.ds(base, BPW)])


def kernel(user, item_i, item_j, embed_user, embed_item):
    return _mfbpr(user.astype(jnp.int32), item_i.astype(jnp.int32),
                  item_j.astype(jnp.int32), embed_user, embed_item)
